# speedup vs baseline: 4.6827x; 4.6827x over previous
import functools

import jax
import jax.numpy as jnp
from jax import lax
from jax.experimental import pallas as pl
from jax.experimental.pallas import tpu as pltpu
from jax.experimental.pallas import tpu_sc as plsc

VOCAB = 100000
EMB = 128
MAX_SEQ = 512
EPS = 1e-12
B = 1024
S = 200
N = B * S

NUM_CORES = 2
NUM_SUBCORES = 16
NW = NUM_CORES * NUM_SUBCORES
TOK_PER_W = N // NW
CHUNK = 128
K_CHUNKS = TOK_PER_W // CHUNK


def _sc_gather(ids_flat, word_table):
    mesh = plsc.VectorSubcoreMesh(core_axis_name="c", subcore_axis_name="s")

    @functools.partial(
        pl.kernel,
        mesh=mesh,
        out_type=jax.ShapeDtypeStruct((N, EMB), jnp.float32),
        scratch_types=[
            pltpu.VMEM((CHUNK,), jnp.int32),
            pltpu.VMEM((CHUNK, EMB), jnp.float32),
            pltpu.SemaphoreType.DMA,
        ],
    )
    def gather_kernel(ids_hbm, table_hbm, out_hbm, idx_v, rows_v, sem):
        wid = lax.axis_index("s") * NUM_CORES + lax.axis_index("c")
        w_base = wid * TOK_PER_W

        def body(k, carry):
            base = w_base + k * CHUNK
            pltpu.sync_copy(ids_hbm.at[pl.ds(base, CHUNK)], idx_v)
            pltpu.async_copy(table_hbm.at[idx_v], rows_v, sem).wait()
            pltpu.sync_copy(rows_v, out_hbm.at[pl.ds(base, CHUNK)])
            return carry

        lax.fori_loop(0, K_CHUNKS, body, 0)

    return gather_kernel(ids_flat, word_table)


G = 8
T = G * S
TC_GRID = B // G


def _tc_ln_kernel(emb_ref, tt_ref, pos_ref, type_ref, gamma_ref, beta_ref, out_ref):
    e = emb_ref[...] + pos_ref[...]
    tt = tt_ref[...]
    t0 = type_ref[0:1, :]
    t1 = type_ref[1:2, :]
    e = e + t0 + tt * (t1 - t0)
    mean = jnp.mean(e, axis=-1, keepdims=True)
    c = e - mean
    var = jnp.mean(c * c, axis=-1, keepdims=True)
    y = c * lax.rsqrt(var + EPS)
    out_ref[...] = y * gamma_ref[...] + beta_ref[...]


def _tc_layernorm(emb, tt_f, pos_tiled, type_table, gamma2, beta2):
    return pl.pallas_call(
        _tc_ln_kernel,
        grid=(TC_GRID,),
        in_specs=[
            pl.BlockSpec((T, EMB), lambda i: (i, 0)),
            pl.BlockSpec((T, 1), lambda i: (i, 0)),
            pl.BlockSpec((T, EMB), lambda i: (0, 0)),
            pl.BlockSpec((2, EMB), lambda i: (0, 0)),
            pl.BlockSpec((1, EMB), lambda i: (0, 0)),
            pl.BlockSpec((1, EMB), lambda i: (0, 0)),
        ],
        out_specs=pl.BlockSpec((T, EMB), lambda i: (i, 0)),
        out_shape=jax.ShapeDtypeStruct((N, EMB), jnp.float32),
    )(emb, tt_f, pos_tiled, type_table, gamma2, beta2)


def kernel(input_ids, token_type_ids, word_table, pos_table, type_table, gamma, beta):
    ids_flat = input_ids.reshape(-1)
    emb = _sc_gather(ids_flat, word_table)

    tt_f = token_type_ids.astype(jnp.float32).reshape(N, 1)
    pos_tiled = jnp.tile(pos_table[:S], (G, 1))
    gamma2 = gamma.reshape(1, EMB)
    beta2 = beta.reshape(1, EMB)
    out = _tc_layernorm(emb, tt_f, pos_tiled, type_table, gamma2, beta2)
    return out.reshape(B, S, EMB)


# speedup vs baseline: 5.2751x; 1.1265x over previous
import functools

import jax
import jax.numpy as jnp
from jax import lax
from jax.experimental import pallas as pl
from jax.experimental.pallas import tpu as pltpu
from jax.experimental.pallas import tpu_sc as plsc

VOCAB = 100000
EMB = 128
MAX_SEQ = 512
EPS = 1e-12
B = 1024
S = 200
N = B * S

NUM_CORES = 2
NUM_SUBCORES = 16
NW = NUM_CORES * NUM_SUBCORES
TOK_PER_W = N // NW
CHUNK = 128
K_CHUNKS = TOK_PER_W // CHUNK


def _sc_gather(ids_flat, word_table):
    mesh = plsc.VectorSubcoreMesh(core_axis_name="c", subcore_axis_name="s")

    @functools.partial(
        pl.kernel,
        mesh=mesh,
        out_type=jax.ShapeDtypeStruct((N, EMB), jnp.float32),
        scratch_types=[
            pltpu.VMEM((TOK_PER_W,), jnp.int32),
            pltpu.VMEM((CHUNK, EMB), jnp.float32),
            pltpu.VMEM((CHUNK, EMB), jnp.float32),
            pltpu.SemaphoreType.DMA,
            pltpu.SemaphoreType.DMA,
            pltpu.SemaphoreType.DMA,
            pltpu.SemaphoreType.DMA,
        ],
    )
    def gather_kernel(ids_hbm, table_hbm, out_hbm, ids_v, rows0, rows1,
                      gsem0, gsem1, ssem0, ssem1):
        wid = lax.axis_index("s") * NUM_CORES + lax.axis_index("c")
        w_base = wid * TOK_PER_W
        rows = (rows0, rows1)
        gsem = (gsem0, gsem1)
        ssem = (ssem0, ssem1)

        def gather_copy(c, b):
            idx = ids_v.at[pl.ds(c * CHUNK, CHUNK)]
            return pltpu.make_async_copy(table_hbm.at[idx], rows[b], gsem[b])

        def store_copy(c, b):
            return pltpu.make_async_copy(
                rows[b], out_hbm.at[pl.ds(w_base + c * CHUNK, CHUNK)], ssem[b])

        pltpu.sync_copy(ids_hbm.at[pl.ds(w_base, TOK_PER_W)], ids_v)

        gather_copy(0, 0).start()
        gather_copy(1, 1).start()
        gather_copy(0, 0).wait()
        store_copy(0, 0).start()

        def pair(k2, carry):
            c = 2 * k2
            store_copy(c - 2, 0).wait()
            gather_copy(c, 0).start()
            gather_copy(c - 1, 1).wait()
            store_copy(c - 1, 1).start()
            store_copy(c - 1, 1).wait()
            gather_copy(c + 1, 1).start()
            gather_copy(c, 0).wait()
            store_copy(c, 0).start()
            return carry

        lax.fori_loop(1, K_CHUNKS // 2, pair, 0)

        gather_copy(K_CHUNKS - 1, 1).wait()
        store_copy(K_CHUNKS - 1, 1).start()
        store_copy(K_CHUNKS - 2, 0).wait()
        store_copy(K_CHUNKS - 1, 1).wait()

    return gather_kernel(ids_flat, word_table)


G = 8
T = G * S
TC_GRID = B // G


def _tc_ln_kernel(emb_ref, tt_ref, pos_ref, type_ref, gamma_ref, beta_ref, out_ref):
    e = emb_ref[...] + pos_ref[...]
    tt = tt_ref[...]
    t0 = type_ref[0:1, :]
    t1 = type_ref[1:2, :]
    e = e + t0 + tt * (t1 - t0)
    mean = jnp.mean(e, axis=-1, keepdims=True)
    c = e - mean
    var = jnp.mean(c * c, axis=-1, keepdims=True)
    y = c * lax.rsqrt(var + EPS)
    out_ref[...] = y * gamma_ref[...] + beta_ref[...]


def _tc_layernorm(emb, tt_f, pos_tiled, type_table, gamma2, beta2):
    return pl.pallas_call(
        _tc_ln_kernel,
        grid=(TC_GRID,),
        in_specs=[
            pl.BlockSpec((T, EMB), lambda i: (i, 0)),
            pl.BlockSpec((T, 1), lambda i: (i, 0)),
            pl.BlockSpec((T, EMB), lambda i: (0, 0)),
            pl.BlockSpec((2, EMB), lambda i: (0, 0)),
            pl.BlockSpec((1, EMB), lambda i: (0, 0)),
            pl.BlockSpec((1, EMB), lambda i: (0, 0)),
        ],
        out_specs=pl.BlockSpec((T, EMB), lambda i: (i, 0)),
        out_shape=jax.ShapeDtypeStruct((N, EMB), jnp.float32),
    )(emb, tt_f, pos_tiled, type_table, gamma2, beta2)


def kernel(input_ids, token_type_ids, word_table, pos_table, type_table, gamma, beta):
    ids_flat = input_ids.reshape(-1)
    emb = _sc_gather(ids_flat, word_table)

    tt_f = token_type_ids.astype(jnp.float32).reshape(N, 1)
    pos_tiled = jnp.tile(pos_table[:S], (G, 1))
    gamma2 = gamma.reshape(1, EMB)
    beta2 = beta.reshape(1, EMB)
    out = _tc_layernorm(emb, tt_f, pos_tiled, type_table, gamma2, beta2)
    return out.reshape(B, S, EMB)
